# baseline (speedup 1.0000x reference)
import functools

import numpy as np
import jax
import jax.numpy as jnp
from jax import lax
from jax.experimental import pallas as pl
from jax.experimental.pallas import tpu as pltpu
from jax.experimental.pallas import tpu_sc as plsc


def _pe_table(seq_len, d_model):
    pos = np.arange(seq_len)[:, None].astype(np.float32)
    i = np.arange(d_model)[None, :].astype(np.float32)
    angle_rates = 1.0 / np.power(
        10000.0, (2.0 * np.floor(i / 2.0)) / np.float32(d_model))
    angle_rads = pos * angle_rates
    pe = np.zeros((seq_len, d_model), dtype=np.float32)
    pe[:, 0::2] = np.sin(angle_rads[:, 0::2])
    pe[:, 1::2] = np.cos(angle_rads[:, 1::2])
    return jnp.asarray(pe)


@functools.lru_cache(maxsize=None)
def _make_sc_lookup(V, D, S, tokens, interpret=False):
    try:
        info = plsc.get_sparse_core_info()
        NC, NS, L = info.num_cores, info.num_subcores, info.num_lanes
    except ValueError:
        NC, NS, L = 2, 16, 16
    NW = NC * NS
    assert tokens % NW == 0
    pwt = tokens // NW
    CT = S * 4
    while pwt % CT != 0:
        CT //= 2
    assert pwt % CT == 0 and CT % S == 0
    G = CT // S
    nchunk = pwt // CT
    SG = 80
    assert CT % SG == 0
    nsub = CT // SG
    assert D % L == 0
    QV = D // L

    mesh = plsc.VectorSubcoreMesh(
        core_axis_name="c", subcore_axis_name="s",
        num_cores=NC, num_subcores=NS)

    @functools.partial(
        pl.kernel,
        out_type=jax.ShapeDtypeStruct((tokens, D), jnp.float32),
        mesh=mesh,
        scratch_types=[
            pltpu.VMEM((CT,), jnp.int32),
            pltpu.VMEM((CT, D), jnp.float32),
            pltpu.VMEM((S, D), jnp.float32),
            pltpu.SemaphoreType.DMA,
        ],
        compiler_params=pltpu.CompilerParams(use_tc_tiling_on_sc=False),
        interpret=interpret,
    )
    def lookup(table_hbm, idx_hbm, pe_hbm, out_hbm, idx_v, rows_v, pe_v, gsem):
        wid = lax.axis_index("s") * NC + lax.axis_index("c")
        base0 = wid * pwt
        pltpu.sync_copy(pe_hbm, pe_v)

        def chunk(j, carry):
            base = base0 + j * CT
            pltpu.sync_copy(idx_hbm.at[pl.ds(base, CT)], idx_v)
            descs = [
                pltpu.async_copy(
                    table_hbm.at[idx_v.at[pl.ds(k * SG, SG)]],
                    rows_v.at[pl.ds(k * SG, SG)],
                    gsem,
                )
                for k in range(nsub)
            ]
            for dsc in descs:
                dsc.wait()

            def posbody(s, c):
                for q in range(QV):
                    pe_vec = pe_v[s, pl.ds(q * L, L)]
                    for g in range(G):
                        r = g * S + s
                        rows_v[r, pl.ds(q * L, L)] = (
                            rows_v[r, pl.ds(q * L, L)] + pe_vec)
                return c

            lax.fori_loop(0, S, posbody, 0)
            pltpu.sync_copy(rows_v, out_hbm.at[pl.ds(base, CT)])
            return carry

        lax.fori_loop(0, nchunk, chunk, 0)

    return lookup


def kernel(inputs, table):
    B, S = inputs.shape
    V, D = table.shape
    idx = inputs.reshape(-1).astype(jnp.int32)
    pe = _pe_table(S, D)
    f = _make_sc_lookup(V, D, S, B * S)
    out = f(table, idx, pe)
    return out.reshape(B, S, D)
